# baseline (speedup 1.0000x reference)
import functools
import math

import numpy as np

import jax
import jax.numpy as jnp
from jax import lax
from jax.experimental import pallas as pl
from jax.experimental.pallas import tpu as pltpu
from jax.experimental.pallas import tpu_sc as plsc

D_MODEL = 768
SEQ = 4096
BATCH = 4
TOKENS = BATCH * SEQ
SCALE = math.sqrt(float(D_MODEL))

_INFO = plsc.get_sparse_core_info()
NUM_WORKERS = _INFO.num_cores * _INFO.num_subcores
TPW = TOKENS // NUM_WORKERS
CB = 32
NCHUNK = TPW // CB
VPR = D_MODEL // 16
NROW = 3


def _pe_over_scale_table(seq_len, d_model):
    pos = np.arange(seq_len, dtype=np.float32)[:, None]
    div = np.exp(
        np.arange(0, d_model, 2, dtype=np.float32)
        * (-math.log(10000.0) / d_model)
    )
    pe = np.zeros((seq_len, d_model), dtype=np.float32)
    pe[:, 0::2] = np.sin(pos * div)
    pe[:, 1::2] = np.cos(pos * div)
    return jnp.asarray(pe / np.float32(SCALE))


def _sc_body(idx_hbm, table_hbm, pe_hbm, out_hbm, idx_v, *scratch):
    rows = scratch[0:NROW]
    gsem = scratch[NROW:2 * NROW]
    psem = scratch[2 * NROW:3 * NROW]
    ssem = scratch[3 * NROW:4 * NROW]

    wid = lax.axis_index("s") * _INFO.num_cores + lax.axis_index("c")
    base = wid * TPW
    pos0 = (wid % (SEQ // TPW)) * TPW
    pltpu.sync_copy(idx_hbm.at[pl.ds(base, TPW)], idx_v)

    def issue_pe(c):
        b = c % NROW
        return pltpu.async_copy(
            pe_hbm.at[pl.ds(pos0 + c * CB, CB)], rows[b], psem[b])

    def issue_gather(c):
        b = c % NROW
        return pltpu.async_copy(
            table_hbm.at[idx_v.at[pl.ds(c * CB, CB)]], rows[b], gsem[b],
            add=True)

    pes = {0: issue_pe(0)}
    pes[0].wait()
    gathers = {0: issue_gather(0)}
    if NCHUNK > 1:
        pes[1] = issue_pe(1)
    stores = {}
    for c in range(NCHUNK):
        if c + 1 < NCHUNK:
            pes.pop(c + 1).wait()
            gathers[c + 1] = issue_gather(c + 1)
        gathers.pop(c).wait()
        rbuf = rows[c % NROW]

        def scale_row(i, carry):
            for j in range(VPR):
                sl = pl.ds(j * 16, 16)
                rbuf[i, sl] = rbuf[i, sl] * SCALE
            return carry

        lax.fori_loop(0, CB, scale_row, 0)
        stores[c] = pltpu.async_copy(
            rbuf, out_hbm.at[pl.ds(base + c * CB, CB)], ssem[c % NROW])
        if c + 2 < NCHUNK:
            if c - 1 >= 0:
                stores.pop(c - 1).wait()
            pes[c + 2] = issue_pe(c + 2)
    for c in sorted(stores):
        stores[c].wait()


def kernel(x, table):
    idx = x.reshape(-1).astype(jnp.int32)
    pe = _pe_over_scale_table(SEQ, D_MODEL)
    mesh = plsc.VectorSubcoreMesh(core_axis_name="c", subcore_axis_name="s")
    scratch = (
        [pltpu.VMEM((TPW,), jnp.int32)]
        + [pltpu.VMEM((CB, D_MODEL), jnp.float32) for _ in range(NROW)]
        + [pltpu.SemaphoreType.DMA for _ in range(3 * NROW)]
    )
    run = functools.partial(
        pl.kernel,
        out_type=jax.ShapeDtypeStruct((TOKENS, D_MODEL), jnp.float32),
        mesh=mesh,
        scratch_types=scratch,
    )(_sc_body)
    out = run(idx, table, pe)
    return out.reshape(BATCH, SEQ, D_MODEL)
